# Initial kernel scaffold; baseline (speedup 1.0000x reference)
import functools

import numpy as np
import jax
import jax.numpy as jnp
from jax import lax
from jax.experimental import pallas as pl
from jax.experimental.pallas import tpu as pltpu
from jax.experimental.pallas import tpu_sc as plsc

_COEFF = (0.02817, 0.28022, 0.50986, 0.18175)
_EXPO = (0.20162, 0.4029, 0.94229, 3.1998)
_EXPO_A = 0.23
_A0 = 0.4685
_R_INNER = 0.5
_R_OUTER = 2.0
_PREFIX = 14.399645478425668

_N_NODES = 100000
_N_EDGES = 6400000

_NS = 16
_NC = 2
_NW = _NC * _NS
_ROWS_PER_TILE = 1568
_NE_PAD = _NW * _ROWS_PER_TILE * 128
_N_ROWS = _NE_PAD // 128
_CHUNK_ROWS = 16
_N_CHUNKS = _ROWS_PER_TILE // _CHUNK_ROWS
_NODES_PAD = 100352
_NODE_SLICE = _NODES_PAD // _NS
_TBL = 8192
_SPK_W = _N_NODES // 4


def _build_tables():
    coeff = np.array(_COEFF, dtype=np.float64)
    expo = np.array(_EXPO, dtype=np.float64)
    s = np.arange(90, dtype=np.float64)
    zp = s ** _EXPO_A
    u = zp[:, None] + zp[None, :]
    u[0, 0] = 1.0
    inv_a = u / _A0
    x = _R_OUTER * inv_a
    e = np.exp(-expo[:, None, None] * x[None])
    phi = (coeff[:, None, None] * e).sum(0)
    dphi = (coeff[:, None, None] * (-expo[:, None, None] * inv_a[None]) * e).sum(0)
    d2phi = (coeff[:, None, None] * (expo[:, None, None] * inv_a[None]) ** 2 * e).sum(0)
    ro = _R_OUTER
    zble = phi / ro
    dz = (1.0 / ro) * (-phi / ro + dphi)
    d2z = (1.0 / ro) * (d2phi - 2.0 * dphi / ro + 2.0 * phi / ro ** 2)
    tc = ro - _R_INNER
    c = -zble + tc / 2.0 * dz - (1.0 / 12.0) * tc ** 2 * d2z
    b4 = (2.0 * dz - tc * d2z) / tc ** 3 / 4.0
    a3 = (-3.0 * dz + tc * d2z) / tc ** 2 / 3.0

    def pad(t):
        out = np.zeros((_TBL,), dtype=np.float32)
        out[: 90 * 90] = t.astype(np.float32).reshape(-1)
        return out

    return pad(inv_a), pad(c), pad(a3), pad(b4)


_T_IA, _T_C, _T_A3, _T_B4 = _build_tables()


def _zbl_body(r_hbm, pf_hbm, ps_hbm, spk_hbm, tia_hbm, tc_hbm, ta3_hbm, tb4_hbm,
              zer_hbm, pout_hbm, partial_hbm,
              spk_v, tia_v, tc_v, ta3_v, tb4_v, r_v, pf_v, ps_v, out_v, atoms_sh):
    cid = lax.axis_index("c")
    sid = lax.axis_index("s")
    wid = cid * _NS + sid
    base_row = wid * _ROWS_PER_TILE

    pltpu.sync_copy(spk_hbm, spk_v)
    pltpu.sync_copy(tia_hbm, tia_v)
    pltpu.sync_copy(tc_hbm, tc_v)
    pltpu.sync_copy(ta3_hbm, ta3_v)
    pltpu.sync_copy(tb4_hbm, tb4_v)
    pltpu.sync_copy(zer_hbm, atoms_sh.at[pl.ds(sid * _NODE_SLICE, _NODE_SLICE)])
    plsc.subcore_barrier()

    c0, c1, c2, c3 = (jnp.float32(v) for v in _COEFF)
    e0, e1, e2, e3 = (jnp.float32(v) for v in _EXPO)

    def vec_body(i, carry):
        row = i // 8
        col = (i % 8) * 16
        pf16 = pf_v[row, pl.ds(col, 16)]
        ps16 = ps_v[row, pl.ds(col, 16)]
        r16 = r_v[row, pl.ds(col, 16)]
        wi = plsc.load_gather(spk_v, [pf16 >> 2])
        wj = plsc.load_gather(spk_v, [ps16 >> 2])
        si = (wi >> ((pf16 & 3) << 3)) & 0xFF
        sj = (wj >> ((ps16 & 3) << 3)) & 0xFF
        pidx = si * 90 + sj
        ia = plsc.load_gather(tia_v, [pidx])
        cc = plsc.load_gather(tc_v, [pidx])
        a3 = plsc.load_gather(ta3_v, [pidx])
        b4 = plsc.load_gather(tb4_v, [pidx])
        zz = jnp.float32(_PREFIX) * si.astype(jnp.float32) * sj.astype(jnp.float32)
        x = r16 * ia
        phi = (c0 * jnp.exp(-e0 * x) + c1 * jnp.exp(-e1 * x)
               + c2 * jnp.exp(-e2 * x) + c3 * jnp.exp(-e3 * x))
        t = r16 - jnp.float32(_R_INNER)
        o2 = zz * (phi / r16 + cc + t * t * t * (a3 + b4 * t))
        o1 = zz * cc
        res = jnp.where(r16 > jnp.float32(_R_OUTER), jnp.float32(0.0),
                        jnp.where(r16 < jnp.float32(_R_INNER), o1, o2))
        out_v[row, pl.ds(col, 16)] = res
        return carry

    def chunk_body(c, carry):
        row0 = base_row + c * _CHUNK_ROWS
        pltpu.sync_copy(r_hbm.at[pl.ds(row0, _CHUNK_ROWS)], r_v)
        pltpu.sync_copy(pf_hbm.at[pl.ds(row0, _CHUNK_ROWS)], pf_v)
        pltpu.sync_copy(ps_hbm.at[pl.ds(row0, _CHUNK_ROWS)], ps_v)
        lax.fori_loop(0, _CHUNK_ROWS * 8, vec_body, 0)
        pltpu.sync_copy(out_v, pout_hbm.at[pl.ds(row0, _CHUNK_ROWS)])
        pltpu.sync_copy(out_v, atoms_sh.at[pf_v], add=True)
        return carry

    lax.fori_loop(0, _N_CHUNKS, chunk_body, 0)
    plsc.subcore_barrier()
    pltpu.sync_copy(atoms_sh.at[pl.ds(sid * _NODE_SLICE, _NODE_SLICE)],
                    partial_hbm.at[cid, pl.ds(sid * _NODE_SLICE, _NODE_SLICE)])


def _combine_body(p_ref, o_ref):
    o_ref[...] = p_ref[0] + p_ref[1]


@jax.jit
def kernel(r, pair_first, pair_second, species):
    npad = _NE_PAD - _N_EDGES
    r2 = jnp.concatenate([r, jnp.full((npad,), 3.0, jnp.float32)]).reshape(_N_ROWS, 128)
    pf2 = jnp.concatenate([pair_first, jnp.zeros((npad,), jnp.int32)]).reshape(_N_ROWS, 128)
    ps2 = jnp.concatenate([pair_second, jnp.zeros((npad,), jnp.int32)]).reshape(_N_ROWS, 128)
    spv = species.reshape(_SPK_W, 4)
    spk = (spv[:, 0] | (spv[:, 1] << 8) | (spv[:, 2] << 16) | (spv[:, 3] << 24)).astype(jnp.int32)
    zer = jnp.zeros((_NODE_SLICE,), jnp.float32)

    mesh = plsc.VectorSubcoreMesh(core_axis_name="c", subcore_axis_name="s",
                                  num_cores=_NC, num_subcores=_NS)
    zbl = pl.kernel(
        _zbl_body,
        out_type=[
            jax.ShapeDtypeStruct((_N_ROWS, 128), jnp.float32),
            jax.ShapeDtypeStruct((_NC, _NODES_PAD), jnp.float32),
        ],
        mesh=mesh,
        scratch_types=[
            pltpu.VMEM((_SPK_W,), jnp.int32),
            pltpu.VMEM((_TBL,), jnp.float32),
            pltpu.VMEM((_TBL,), jnp.float32),
            pltpu.VMEM((_TBL,), jnp.float32),
            pltpu.VMEM((_TBL,), jnp.float32),
            pltpu.VMEM((_CHUNK_ROWS, 128), jnp.float32),
            pltpu.VMEM((_CHUNK_ROWS, 128), jnp.int32),
            pltpu.VMEM((_CHUNK_ROWS, 128), jnp.int32),
            pltpu.VMEM((_CHUNK_ROWS, 128), jnp.float32),
            pltpu.VMEM_SHARED((_NODES_PAD,), jnp.float32),
        ],
    )
    pair2, partial = zbl(r2, pf2, ps2, spk,
                         jnp.asarray(_T_IA), jnp.asarray(_T_C),
                         jnp.asarray(_T_A3), jnp.asarray(_T_B4), zer)

    atom_pad = pl.pallas_call(
        _combine_body,
        out_shape=jax.ShapeDtypeStruct((784, 128), jnp.float32),
    )(partial.reshape(_NC, 784, 128))

    pair_output = pair2.reshape(-1)[:_N_EDGES]
    atom_output = atom_pad.reshape(-1)[:_N_NODES]
    return (pair_output, atom_output)


# speedup vs baseline: 142.4384x; 142.4384x over previous
import functools

import numpy as np
import jax
import jax.numpy as jnp
from jax import lax
from jax.experimental import pallas as pl
from jax.experimental.pallas import tpu as pltpu
from jax.experimental.pallas import tpu_sc as plsc

_COEFF = (0.02817, 0.28022, 0.50986, 0.18175)
_EXPO = (0.20162, 0.4029, 0.94229, 3.1998)
_EXPO_A = 0.23
_A0 = 0.4685
_R_INNER = 0.5
_R_OUTER = 2.0
_PREFIX = 14.399645478425668

_N_NODES = 100000
_N_EDGES = 6400000

_NS = 16
_NC = 2
_NW = _NC * _NS
_ROWS_PER_TILE = 1568
_NE_PAD = _NW * _ROWS_PER_TILE * 128
_N_ROWS = _NE_PAD // 128
_CHUNK_ROWS = 16
_N_CHUNKS = _ROWS_PER_TILE // _CHUNK_ROWS
_NODES_PAD = 100352
_NODE_SLICE = _NODES_PAD // _NS
_TBL = 8192
_SPK_W = _N_NODES // 4


def _build_tables():
    coeff = np.array(_COEFF, dtype=np.float64)
    expo = np.array(_EXPO, dtype=np.float64)
    s = np.arange(90, dtype=np.float64)
    zp = s ** _EXPO_A
    u = zp[:, None] + zp[None, :]
    u[0, 0] = 1.0
    inv_a = u / _A0
    x = _R_OUTER * inv_a
    e = np.exp(-expo[:, None, None] * x[None])
    phi = (coeff[:, None, None] * e).sum(0)
    dphi = (coeff[:, None, None] * (-expo[:, None, None] * inv_a[None]) * e).sum(0)
    d2phi = (coeff[:, None, None] * (expo[:, None, None] * inv_a[None]) ** 2 * e).sum(0)
    ro = _R_OUTER
    zble = phi / ro
    dz = (1.0 / ro) * (-phi / ro + dphi)
    d2z = (1.0 / ro) * (d2phi - 2.0 * dphi / ro + 2.0 * phi / ro ** 2)
    tc = ro - _R_INNER
    c = -zble + tc / 2.0 * dz - (1.0 / 12.0) * tc ** 2 * d2z
    b4 = (2.0 * dz - tc * d2z) / tc ** 3 / 4.0
    a3 = (-3.0 * dz + tc * d2z) / tc ** 2 / 3.0

    def pad(t):
        out = np.zeros((_TBL,), dtype=np.float32)
        out[: 90 * 90] = t.astype(np.float32).reshape(-1)
        return out

    return pad(inv_a), pad(c), pad(a3), pad(b4)


_T_IA, _T_C, _T_A3, _T_B4 = _build_tables()


def _zbl_body(r_hbm, pf_hbm, ps_hbm, spk_hbm, tia_hbm, tc_hbm, ta3_hbm, tb4_hbm,
              zer_hbm, pout_hbm, partial_hbm,
              spk_v, tia_v, tc_v, ta3_v, tb4_v, r_v, pf_v, ps_v, out_v, atoms_sh):
    cid = lax.axis_index("c")
    sid = lax.axis_index("s")
    wid = cid * _NS + sid
    base_row = wid * _ROWS_PER_TILE

    pltpu.sync_copy(spk_hbm, spk_v)
    pltpu.sync_copy(tia_hbm, tia_v)
    pltpu.sync_copy(tc_hbm, tc_v)
    pltpu.sync_copy(ta3_hbm, ta3_v)
    pltpu.sync_copy(tb4_hbm, tb4_v)
    pltpu.sync_copy(zer_hbm, atoms_sh.at[pl.ds(sid * _NODE_SLICE, _NODE_SLICE)])
    plsc.subcore_barrier()

    c0, c1, c2, c3 = (jnp.float32(v) for v in _COEFF)
    e0, e1, e2, e3 = (jnp.float32(v) for v in _EXPO)

    def vec_body(i, carry):
        row = i // 8
        col = (i % 8) * 16
        pf16 = pf_v[row, pl.ds(col, 16)]
        ps16 = ps_v[row, pl.ds(col, 16)]
        r16 = r_v[row, pl.ds(col, 16)]
        wi = plsc.load_gather(spk_v, [pf16 >> 2])
        wj = plsc.load_gather(spk_v, [ps16 >> 2])
        si = (wi >> ((pf16 & 3) << 3)) & 0xFF
        sj = (wj >> ((ps16 & 3) << 3)) & 0xFF
        pidx = si * 90 + sj
        ia = plsc.load_gather(tia_v, [pidx])
        cc = plsc.load_gather(tc_v, [pidx])
        a3 = plsc.load_gather(ta3_v, [pidx])
        b4 = plsc.load_gather(tb4_v, [pidx])
        zz = jnp.float32(_PREFIX) * si.astype(jnp.float32) * sj.astype(jnp.float32)
        x = r16 * ia
        phi = (c0 * jnp.exp(-e0 * x) + c1 * jnp.exp(-e1 * x)
               + c2 * jnp.exp(-e2 * x) + c3 * jnp.exp(-e3 * x))
        t = r16 - jnp.float32(_R_INNER)
        o2 = zz * (phi / r16 + cc + t * t * t * (a3 + b4 * t))
        o1 = zz * cc
        res = jnp.where(r16 > jnp.float32(_R_OUTER), jnp.float32(0.0),
                        jnp.where(r16 < jnp.float32(_R_INNER), o1, o2))
        out_v[row, pl.ds(col, 16)] = res
        return carry

    def chunk_body(c, carry):
        row0 = base_row + c * _CHUNK_ROWS
        pltpu.sync_copy(r_hbm.at[pl.ds(row0, _CHUNK_ROWS)], r_v)
        pltpu.sync_copy(pf_hbm.at[pl.ds(row0, _CHUNK_ROWS)], pf_v)
        pltpu.sync_copy(ps_hbm.at[pl.ds(row0, _CHUNK_ROWS)], ps_v)
        lax.fori_loop(0, _CHUNK_ROWS * 8, vec_body, 0)
        pltpu.sync_copy(out_v, pout_hbm.at[pl.ds(row0, _CHUNK_ROWS)])
        for j in range(_CHUNK_ROWS):
            pltpu.sync_copy(out_v.at[j], atoms_sh.at[pf_v.at[j]], add=True)
        return carry

    lax.fori_loop(0, _N_CHUNKS, chunk_body, 0)
    plsc.subcore_barrier()
    pltpu.sync_copy(atoms_sh.at[pl.ds(sid * _NODE_SLICE, _NODE_SLICE)],
                    partial_hbm.at[cid, pl.ds(sid * _NODE_SLICE, _NODE_SLICE)])


def _combine_body(p_ref, o_ref):
    o_ref[...] = p_ref[0] + p_ref[1]


@jax.jit
def kernel(r, pair_first, pair_second, species):
    npad = _NE_PAD - _N_EDGES
    r2 = jnp.concatenate([r, jnp.full((npad,), 3.0, jnp.float32)]).reshape(_N_ROWS, 128)
    pf2 = jnp.concatenate([pair_first, jnp.zeros((npad,), jnp.int32)]).reshape(_N_ROWS, 128)
    ps2 = jnp.concatenate([pair_second, jnp.zeros((npad,), jnp.int32)]).reshape(_N_ROWS, 128)
    spv = species.reshape(_SPK_W, 4)
    spk = (spv[:, 0] | (spv[:, 1] << 8) | (spv[:, 2] << 16) | (spv[:, 3] << 24)).astype(jnp.int32)
    zer = jnp.zeros((_NODE_SLICE,), jnp.float32)

    mesh = plsc.VectorSubcoreMesh(core_axis_name="c", subcore_axis_name="s",
                                  num_cores=_NC, num_subcores=_NS)
    zbl = pl.kernel(
        _zbl_body,
        out_type=[
            jax.ShapeDtypeStruct((_N_ROWS, 128), jnp.float32),
            jax.ShapeDtypeStruct((_NC, _NODES_PAD), jnp.float32),
        ],
        mesh=mesh,
        compiler_params=pltpu.CompilerParams(needs_layout_passes=False),
        scratch_types=[
            pltpu.VMEM((_SPK_W,), jnp.int32),
            pltpu.VMEM((_TBL,), jnp.float32),
            pltpu.VMEM((_TBL,), jnp.float32),
            pltpu.VMEM((_TBL,), jnp.float32),
            pltpu.VMEM((_TBL,), jnp.float32),
            pltpu.VMEM((_CHUNK_ROWS, 128), jnp.float32),
            pltpu.VMEM((_CHUNK_ROWS, 128), jnp.int32),
            pltpu.VMEM((_CHUNK_ROWS, 128), jnp.int32),
            pltpu.VMEM((_CHUNK_ROWS, 128), jnp.float32),
            pltpu.VMEM_SHARED((_NODES_PAD,), jnp.float32),
        ],
    )
    pair2, partial = zbl(r2, pf2, ps2, spk,
                         jnp.asarray(_T_IA), jnp.asarray(_T_C),
                         jnp.asarray(_T_A3), jnp.asarray(_T_B4), zer)

    atom_pad = pl.pallas_call(
        _combine_body,
        out_shape=jax.ShapeDtypeStruct((784, 128), jnp.float32),
    )(partial.reshape(_NC, 784, 128))

    pair_output = pair2.reshape(-1)[:_N_EDGES]
    atom_output = atom_pad.reshape(-1)[:_N_NODES]
    return (pair_output, atom_output)


# speedup vs baseline: 153.7526x; 1.0794x over previous
import functools

import numpy as np
import jax
import jax.numpy as jnp
from jax import lax
from jax.experimental import pallas as pl
from jax.experimental.pallas import tpu as pltpu
from jax.experimental.pallas import tpu_sc as plsc

_COEFF = (0.02817, 0.28022, 0.50986, 0.18175)
_EXPO = (0.20162, 0.4029, 0.94229, 3.1998)
_EXPO_A = 0.23
_A0 = 0.4685
_R_INNER = 0.5
_R_OUTER = 2.0
_PREFIX = 14.399645478425668

_N_NODES = 100000
_N_EDGES = 6400000

_NS = 16
_NC = 2
_NW = _NC * _NS
_EDGES_PER_TILE = 200704
_NE_PAD = _NW * _EDGES_PER_TILE
_CHUNK = 2048
_N_CHUNKS = _EDGES_PER_TILE // _CHUNK
_NODES_PAD = 100352
_NODE_SLICE = _NODES_PAD // _NS
_TBL = 8192
_SPK_W = _N_NODES // 4


def _build_tables():
    coeff = np.array(_COEFF, dtype=np.float64)
    expo = np.array(_EXPO, dtype=np.float64)
    s = np.arange(90, dtype=np.float64)
    zp = s ** _EXPO_A
    u = zp[:, None] + zp[None, :]
    u[0, 0] = 1.0
    inv_a = u / _A0
    x = _R_OUTER * inv_a
    e = np.exp(-expo[:, None, None] * x[None])
    phi = (coeff[:, None, None] * e).sum(0)
    dphi = (coeff[:, None, None] * (-expo[:, None, None] * inv_a[None]) * e).sum(0)
    d2phi = (coeff[:, None, None] * (expo[:, None, None] * inv_a[None]) ** 2 * e).sum(0)
    ro = _R_OUTER
    zble = phi / ro
    dz = (1.0 / ro) * (-phi / ro + dphi)
    d2z = (1.0 / ro) * (d2phi - 2.0 * dphi / ro + 2.0 * phi / ro ** 2)
    tc = ro - _R_INNER
    c = -zble + tc / 2.0 * dz - (1.0 / 12.0) * tc ** 2 * d2z
    b4 = (2.0 * dz - tc * d2z) / tc ** 3 / 4.0
    a3 = (-3.0 * dz + tc * d2z) / tc ** 2 / 3.0

    def pad(t):
        out = np.zeros((_TBL,), dtype=np.float32)
        out[: 90 * 90] = t.astype(np.float32).reshape(-1)
        return out

    return pad(inv_a), pad(c), pad(a3), pad(b4)


_T_IA, _T_C, _T_A3, _T_B4 = _build_tables()


def _zbl_body(r_hbm, pf_hbm, ps_hbm, spk_hbm, tia_hbm, tc_hbm, ta3_hbm, tb4_hbm,
              zer_hbm, pout_hbm, partial_hbm,
              spk_v, tia_v, tc_v, ta3_v, tb4_v, r_v, pf_v, ps_v, out_v, atoms_sh):
    cid = lax.axis_index("c")
    sid = lax.axis_index("s")
    wid = cid * _NS + sid
    tile_base = wid * _EDGES_PER_TILE

    pltpu.sync_copy(spk_hbm, spk_v)
    pltpu.sync_copy(tia_hbm, tia_v)
    pltpu.sync_copy(tc_hbm, tc_v)
    pltpu.sync_copy(ta3_hbm, ta3_v)
    pltpu.sync_copy(tb4_hbm, tb4_v)
    pltpu.sync_copy(zer_hbm, atoms_sh.at[pl.ds(sid * _NODE_SLICE, _NODE_SLICE)])
    plsc.subcore_barrier()

    c0, c1, c2, c3 = (jnp.float32(v) for v in _COEFF)
    e0, e1, e2, e3 = (jnp.float32(v) for v in _EXPO)

    def vec_compute(off):
        pf16 = pf_v[pl.ds(off, 16)]
        ps16 = ps_v[pl.ds(off, 16)]
        r16 = r_v[pl.ds(off, 16)]
        wi = plsc.load_gather(spk_v, [pf16 >> 2])
        wj = plsc.load_gather(spk_v, [ps16 >> 2])
        si = (wi >> ((pf16 & 3) << 3)) & 0xFF
        sj = (wj >> ((ps16 & 3) << 3)) & 0xFF
        pidx = si * 90 + sj
        ia = plsc.load_gather(tia_v, [pidx])
        cc = plsc.load_gather(tc_v, [pidx])
        a3 = plsc.load_gather(ta3_v, [pidx])
        b4 = plsc.load_gather(tb4_v, [pidx])
        zz = jnp.float32(_PREFIX) * si.astype(jnp.float32) * sj.astype(jnp.float32)
        x = r16 * ia
        phi = (c0 * jnp.exp(-e0 * x) + c1 * jnp.exp(-e1 * x)
               + c2 * jnp.exp(-e2 * x) + c3 * jnp.exp(-e3 * x))
        t = r16 - jnp.float32(_R_INNER)
        o2 = zz * (phi / r16 + cc + t * t * t * (a3 + b4 * t))
        o1 = zz * cc
        res = jnp.where(r16 > jnp.float32(_R_OUTER), jnp.float32(0.0),
                        jnp.where(r16 < jnp.float32(_R_INNER), o1, o2))
        out_v[pl.ds(off, 16)] = res

    def vec_body(i, carry):
        base = i * 128
        for u in range(8):
            vec_compute(base + u * 16)
        return carry

    def chunk_body(c, carry):
        base = tile_base + c * _CHUNK
        pltpu.sync_copy(r_hbm.at[pl.ds(base, _CHUNK)], r_v)
        pltpu.sync_copy(pf_hbm.at[pl.ds(base, _CHUNK)], pf_v)
        pltpu.sync_copy(ps_hbm.at[pl.ds(base, _CHUNK)], ps_v)
        lax.fori_loop(0, _CHUNK // 128, vec_body, 0)
        pltpu.sync_copy(out_v, pout_hbm.at[pl.ds(base, _CHUNK)])
        pltpu.sync_copy(out_v, atoms_sh.at[pf_v], add=True)
        return carry

    lax.fori_loop(0, _N_CHUNKS, chunk_body, 0)
    plsc.subcore_barrier()
    pltpu.sync_copy(atoms_sh.at[pl.ds(sid * _NODE_SLICE, _NODE_SLICE)],
                    partial_hbm.at[cid, pl.ds(sid * _NODE_SLICE, _NODE_SLICE)])


def _combine_body(p_ref, o_ref):
    o_ref[...] = p_ref[0] + p_ref[1]


@jax.jit
def kernel(r, pair_first, pair_second, species):
    npad = _NE_PAD - _N_EDGES
    r2 = jnp.concatenate([r, jnp.full((npad,), 3.0, jnp.float32)])
    pf2 = jnp.concatenate([pair_first, jnp.zeros((npad,), jnp.int32)])
    ps2 = jnp.concatenate([pair_second, jnp.zeros((npad,), jnp.int32)])
    spv = species.reshape(_SPK_W, 4)
    spk = (spv[:, 0] | (spv[:, 1] << 8) | (spv[:, 2] << 16) | (spv[:, 3] << 24)).astype(jnp.int32)
    zer = jnp.zeros((_NODE_SLICE,), jnp.float32)

    mesh = plsc.VectorSubcoreMesh(core_axis_name="c", subcore_axis_name="s",
                                  num_cores=_NC, num_subcores=_NS)
    zbl = pl.kernel(
        _zbl_body,
        out_type=[
            jax.ShapeDtypeStruct((_NE_PAD,), jnp.float32),
            jax.ShapeDtypeStruct((_NC, _NODES_PAD), jnp.float32),
        ],
        mesh=mesh,
        compiler_params=pltpu.CompilerParams(needs_layout_passes=False),
        scratch_types=[
            pltpu.VMEM((_SPK_W,), jnp.int32),
            pltpu.VMEM((_TBL,), jnp.float32),
            pltpu.VMEM((_TBL,), jnp.float32),
            pltpu.VMEM((_TBL,), jnp.float32),
            pltpu.VMEM((_TBL,), jnp.float32),
            pltpu.VMEM((_CHUNK,), jnp.float32),
            pltpu.VMEM((_CHUNK,), jnp.int32),
            pltpu.VMEM((_CHUNK,), jnp.int32),
            pltpu.VMEM((_CHUNK,), jnp.float32),
            pltpu.VMEM_SHARED((_NODES_PAD,), jnp.float32),
        ],
    )
    pair2, partial = zbl(r2, pf2, ps2, spk,
                         jnp.asarray(_T_IA), jnp.asarray(_T_C),
                         jnp.asarray(_T_A3), jnp.asarray(_T_B4), zer)

    atom_pad = pl.pallas_call(
        _combine_body,
        out_shape=jax.ShapeDtypeStruct((784, 128), jnp.float32),
    )(partial.reshape(_NC, 784, 128))

    pair_output = pair2[:_N_EDGES]
    atom_output = atom_pad.reshape(-1)[:_N_NODES]
    return (pair_output, atom_output)


# speedup vs baseline: 193.9220x; 1.2613x over previous
import functools

import numpy as np
import jax
import jax.numpy as jnp
from jax import lax
from jax.experimental import pallas as pl
from jax.experimental.pallas import tpu as pltpu
from jax.experimental.pallas import tpu_sc as plsc

_COEFF = (0.02817, 0.28022, 0.50986, 0.18175)
_EXPO = (0.20162, 0.4029, 0.94229, 3.1998)
_EXPO_A = 0.23
_A0 = 0.4685
_R_INNER = 0.5
_R_OUTER = 2.0
_PREFIX = 14.399645478425668

_N_NODES = 100000
_N_EDGES = 6400000

_NS = 16
_NC = 2
_NW = _NC * _NS
_EDGES_PER_TILE = 200704
_NE_PAD = _NW * _EDGES_PER_TILE
_CHUNK = 2048
_N_CHUNKS = _EDGES_PER_TILE // _CHUNK
_NODES_PAD = 100352
_NODE_SLICE = _NODES_PAD // _NS
_TBL = 8192
_SPK_W = _N_NODES // 4


def _build_tables():
    coeff = np.array(_COEFF, dtype=np.float64)
    expo = np.array(_EXPO, dtype=np.float64)
    s = np.arange(90, dtype=np.float64)
    zp = s ** _EXPO_A
    u = zp[:, None] + zp[None, :]
    u[0, 0] = 1.0
    inv_a = u / _A0
    x = _R_OUTER * inv_a
    e = np.exp(-expo[:, None, None] * x[None])
    phi = (coeff[:, None, None] * e).sum(0)
    dphi = (coeff[:, None, None] * (-expo[:, None, None] * inv_a[None]) * e).sum(0)
    d2phi = (coeff[:, None, None] * (expo[:, None, None] * inv_a[None]) ** 2 * e).sum(0)
    ro = _R_OUTER
    zble = phi / ro
    dz = (1.0 / ro) * (-phi / ro + dphi)
    d2z = (1.0 / ro) * (d2phi - 2.0 * dphi / ro + 2.0 * phi / ro ** 2)
    tc = ro - _R_INNER
    c = -zble + tc / 2.0 * dz - (1.0 / 12.0) * tc ** 2 * d2z
    b4 = (2.0 * dz - tc * d2z) / tc ** 3 / 4.0
    a3 = (-3.0 * dz + tc * d2z) / tc ** 2 / 3.0

    def pad(t):
        out = np.zeros((_TBL,), dtype=np.float32)
        out[: 90 * 90] = t.astype(np.float32).reshape(-1)
        return out

    return pad(inv_a), pad(c), pad(a3), pad(b4)


_T_IA, _T_C, _T_A3, _T_B4 = _build_tables()


def _zbl_body(r_hbm, pf_hbm, ps_hbm, spk_hbm, tia_hbm, tc_hbm, ta3_hbm, tb4_hbm,
              zer_hbm, pout_hbm, partial_hbm,
              spk_v, tia_v, tc_v, ta3_v, tb4_v,
              r_v0, pf_v0, ps_v0, out_v0, r_v1, pf_v1, ps_v1, out_v1, atoms_sh,
              sem_in0, sem_in1, sem_out0, sem_out1):
    cid = lax.axis_index("c")
    sid = lax.axis_index("s")
    wid = cid * _NS + sid
    tile_base = wid * _EDGES_PER_TILE

    pltpu.sync_copy(spk_hbm, spk_v)
    pltpu.sync_copy(tia_hbm, tia_v)
    pltpu.sync_copy(tc_hbm, tc_v)
    pltpu.sync_copy(ta3_hbm, ta3_v)
    pltpu.sync_copy(tb4_hbm, tb4_v)
    pltpu.sync_copy(zer_hbm, atoms_sh.at[pl.ds(sid * _NODE_SLICE, _NODE_SLICE)])
    plsc.subcore_barrier()

    c0, c1, c2, c3 = (jnp.float32(v) for v in _COEFF)
    e0, e1, e2, e3 = (jnp.float32(v) for v in _EXPO)

    slots = ((r_v0, pf_v0, ps_v0, out_v0, sem_in0, sem_out0),
             (r_v1, pf_v1, ps_v1, out_v1, sem_in1, sem_out1))

    def start_in(c, sl):
        rv, pfv, psv, _, si, _ = slots[sl]
        base = tile_base + c * _CHUNK
        pltpu.async_copy(r_hbm.at[pl.ds(base, _CHUNK)], rv, si)
        pltpu.async_copy(pf_hbm.at[pl.ds(base, _CHUNK)], pfv, si)
        pltpu.async_copy(ps_hbm.at[pl.ds(base, _CHUNK)], psv, si)

    def wait_in(sl):
        rv, pfv, psv, _, si, _ = slots[sl]
        pltpu.make_async_copy(r_hbm.at[pl.ds(0, _CHUNK)], rv, si).wait()
        pltpu.make_async_copy(pf_hbm.at[pl.ds(0, _CHUNK)], pfv, si).wait()
        pltpu.make_async_copy(ps_hbm.at[pl.ds(0, _CHUNK)], psv, si).wait()

    def outscat(c, sl):
        _, pfv, _, ov, _, so = slots[sl]
        base = tile_base + c * _CHUNK
        pltpu.sync_copy(ov, pout_hbm.at[pl.ds(base, _CHUNK)])
        pltpu.sync_copy(ov, atoms_sh.at[pfv], add=True)

    def vec_compute(sl, off):
        rv, pfv, psv, ov, _, _ = slots[sl]
        pf16 = pfv[pl.ds(off, 16)]
        ps16 = psv[pl.ds(off, 16)]
        r16 = rv[pl.ds(off, 16)]
        wi = plsc.load_gather(spk_v, [pf16 >> 2])
        wj = plsc.load_gather(spk_v, [ps16 >> 2])
        si = (wi >> ((pf16 & 3) << 3)) & 0xFF
        sj = (wj >> ((ps16 & 3) << 3)) & 0xFF
        pidx = si * 90 + sj
        ia = plsc.load_gather(tia_v, [pidx])
        cc = plsc.load_gather(tc_v, [pidx])
        a3 = plsc.load_gather(ta3_v, [pidx])
        b4 = plsc.load_gather(tb4_v, [pidx])
        zz = jnp.float32(_PREFIX) * si.astype(jnp.float32) * sj.astype(jnp.float32)
        x = r16 * ia
        phi = (c0 * jnp.exp(-e0 * x) + c1 * jnp.exp(-e1 * x)
               + c2 * jnp.exp(-e2 * x) + c3 * jnp.exp(-e3 * x))
        t = r16 - jnp.float32(_R_INNER)
        o2 = zz * (phi / r16 + cc + t * t * t * (a3 + b4 * t))
        o1 = zz * cc
        res = jnp.where(r16 > jnp.float32(_R_OUTER), jnp.float32(0.0),
                        jnp.where(r16 < jnp.float32(_R_INNER), o1, o2))
        ov[pl.ds(off, 16)] = res

    def compute(sl):
        def vec_body(i, carry):
            base = i * 128
            for u in range(8):
                vec_compute(sl, base + u * 16)
            return carry
        lax.fori_loop(0, _CHUNK // 128, vec_body, 0)

    def pair_body(g, carry):
        a = 2 * g
        start_in(a + 1, 1)
        wait_in(0)
        compute(0)
        outscat(a, 0)

        @pl.when(g < _N_CHUNKS // 2 - 1)
        def _():
            start_in(a + 2, 0)
        wait_in(1)
        compute(1)
        outscat(a + 1, 1)
        return carry

    start_in(0, 0)
    lax.fori_loop(0, _N_CHUNKS // 2, pair_body, 0)
    plsc.subcore_barrier()
    pltpu.sync_copy(atoms_sh.at[pl.ds(sid * _NODE_SLICE, _NODE_SLICE)],
                    partial_hbm.at[cid, pl.ds(sid * _NODE_SLICE, _NODE_SLICE)])


def _combine_body(p_ref, o_ref):
    o_ref[...] = p_ref[0] + p_ref[1]


@jax.jit
def kernel(r, pair_first, pair_second, species):
    npad = _NE_PAD - _N_EDGES
    r2 = jnp.concatenate([r, jnp.full((npad,), 3.0, jnp.float32)])
    pf2 = jnp.concatenate([pair_first, jnp.zeros((npad,), jnp.int32)])
    ps2 = jnp.concatenate([pair_second, jnp.zeros((npad,), jnp.int32)])
    spv = species.reshape(_SPK_W, 4)
    spk = (spv[:, 0] | (spv[:, 1] << 8) | (spv[:, 2] << 16) | (spv[:, 3] << 24)).astype(jnp.int32)
    zer = jnp.zeros((_NODE_SLICE,), jnp.float32)

    mesh = plsc.VectorSubcoreMesh(core_axis_name="c", subcore_axis_name="s",
                                  num_cores=_NC, num_subcores=_NS)
    zbl = pl.kernel(
        _zbl_body,
        out_type=[
            jax.ShapeDtypeStruct((_NE_PAD,), jnp.float32),
            jax.ShapeDtypeStruct((_NC, _NODES_PAD), jnp.float32),
        ],
        mesh=mesh,
        compiler_params=pltpu.CompilerParams(needs_layout_passes=False),
        scratch_types=[
            pltpu.VMEM((_SPK_W,), jnp.int32),
            pltpu.VMEM((_TBL,), jnp.float32),
            pltpu.VMEM((_TBL,), jnp.float32),
            pltpu.VMEM((_TBL,), jnp.float32),
            pltpu.VMEM((_TBL,), jnp.float32),
            pltpu.VMEM((_CHUNK,), jnp.float32),
            pltpu.VMEM((_CHUNK,), jnp.int32),
            pltpu.VMEM((_CHUNK,), jnp.int32),
            pltpu.VMEM((_CHUNK,), jnp.float32),
            pltpu.VMEM((_CHUNK,), jnp.float32),
            pltpu.VMEM((_CHUNK,), jnp.int32),
            pltpu.VMEM((_CHUNK,), jnp.int32),
            pltpu.VMEM((_CHUNK,), jnp.float32),
            pltpu.VMEM_SHARED((_NODES_PAD,), jnp.float32),
            pltpu.SemaphoreType.DMA,
            pltpu.SemaphoreType.DMA,
            pltpu.SemaphoreType.DMA,
            pltpu.SemaphoreType.DMA,
        ],
    )
    pair2, partial = zbl(r2, pf2, ps2, spk,
                         jnp.asarray(_T_IA), jnp.asarray(_T_C),
                         jnp.asarray(_T_A3), jnp.asarray(_T_B4), zer)

    atom_pad = pl.pallas_call(
        _combine_body,
        out_shape=jax.ShapeDtypeStruct((784, 128), jnp.float32),
    )(partial.reshape(_NC, 784, 128))

    pair_output = pair2[:_N_EDGES]
    atom_output = atom_pad.reshape(-1)[:_N_NODES]
    return (pair_output, atom_output)


# speedup vs baseline: 206.8465x; 1.0666x over previous
import functools

import numpy as np
import jax
import jax.numpy as jnp
from jax import lax
from jax.experimental import pallas as pl
from jax.experimental.pallas import tpu as pltpu
from jax.experimental.pallas import tpu_sc as plsc

_COEFF = (0.02817, 0.28022, 0.50986, 0.18175)
_EXPO = (0.20162, 0.4029, 0.94229, 3.1998)
_EXPO_A = 0.23
_A0 = 0.4685
_R_INNER = 0.5
_R_OUTER = 2.0
_PREFIX = 14.399645478425668

_N_NODES = 100000
_N_EDGES = 6400000

_NS = 16
_NC = 2
_NW = _NC * _NS
_EDGES_PER_TILE = 200704
_NE_PAD = _NW * _EDGES_PER_TILE
_CHUNK = 2048
_N_CHUNKS = _EDGES_PER_TILE // _CHUNK
_NODES_PAD = 100352
_NODE_SLICE = _NODES_PAD // _NS
_TBL = 8192
_SPK_W = _N_NODES // 4


def _build_tables():
    coeff = np.array(_COEFF, dtype=np.float64)
    expo = np.array(_EXPO, dtype=np.float64)
    s = np.arange(90, dtype=np.float64)
    zp = s ** _EXPO_A
    u = zp[:, None] + zp[None, :]
    u[0, 0] = 1.0
    inv_a = u / _A0
    x = _R_OUTER * inv_a
    e = np.exp(-expo[:, None, None] * x[None])
    phi = (coeff[:, None, None] * e).sum(0)
    dphi = (coeff[:, None, None] * (-expo[:, None, None] * inv_a[None]) * e).sum(0)
    d2phi = (coeff[:, None, None] * (expo[:, None, None] * inv_a[None]) ** 2 * e).sum(0)
    ro = _R_OUTER
    zble = phi / ro
    dz = (1.0 / ro) * (-phi / ro + dphi)
    d2z = (1.0 / ro) * (d2phi - 2.0 * dphi / ro + 2.0 * phi / ro ** 2)
    tc = ro - _R_INNER
    c = -zble + tc / 2.0 * dz - (1.0 / 12.0) * tc ** 2 * d2z
    b4 = (2.0 * dz - tc * d2z) / tc ** 3 / 4.0
    a3 = (-3.0 * dz + tc * d2z) / tc ** 2 / 3.0

    def pad(t):
        out = np.zeros((_TBL,), dtype=np.float32)
        out[: 90 * 90] = t.astype(np.float32).reshape(-1)
        return out

    return pad(inv_a), pad(c), pad(a3), pad(b4)


_T_IA, _T_C, _T_A3, _T_B4 = _build_tables()


def _zbl_body(r_hbm, pf_hbm, ps_hbm, spk_hbm, tia_hbm, tc_hbm, ta3_hbm, tb4_hbm,
              zer_hbm, pout_hbm, partial_hbm,
              spk_v, tia_v, tc_v, ta3_v, tb4_v,
              r_v0, pf_v0, ps_v0, out_v0, r_v1, pf_v1, ps_v1, out_v1, atoms_sh,
              sem_in0, sem_in1, sem_po0, sem_po1, sem_sc):
    cid = lax.axis_index("c")
    sid = lax.axis_index("s")
    wid = cid * _NS + sid
    tile_base = wid * _EDGES_PER_TILE

    pltpu.sync_copy(spk_hbm, spk_v)
    pltpu.sync_copy(tia_hbm, tia_v)
    pltpu.sync_copy(tc_hbm, tc_v)
    pltpu.sync_copy(ta3_hbm, ta3_v)
    pltpu.sync_copy(tb4_hbm, tb4_v)
    pltpu.sync_copy(zer_hbm, atoms_sh.at[pl.ds(sid * _NODE_SLICE, _NODE_SLICE)])
    plsc.subcore_barrier()

    c0, c1, c2, c3 = (jnp.float32(v) for v in _COEFF)
    e0, e1, e2, e3 = (jnp.float32(v) for v in _EXPO)

    slots = ((r_v0, pf_v0, ps_v0, out_v0, sem_in0, sem_po0),
             (r_v1, pf_v1, ps_v1, out_v1, sem_in1, sem_po1))

    def start_in(c, sl):
        rv, pfv, psv, _, si, _ = slots[sl]
        base = tile_base + c * _CHUNK
        pltpu.async_copy(r_hbm.at[pl.ds(base, _CHUNK)], rv, si)
        pltpu.async_copy(pf_hbm.at[pl.ds(base, _CHUNK)], pfv, si)
        pltpu.async_copy(ps_hbm.at[pl.ds(base, _CHUNK)], psv, si)

    def wait_in(sl):
        rv, pfv, psv, _, si, _ = slots[sl]
        pltpu.make_async_copy(r_hbm.at[pl.ds(0, _CHUNK)], rv, si).wait()
        pltpu.make_async_copy(pf_hbm.at[pl.ds(0, _CHUNK)], pfv, si).wait()
        pltpu.make_async_copy(ps_hbm.at[pl.ds(0, _CHUNK)], psv, si).wait()

    def start_pout(c, sl):
        _, _, _, ov, _, so = slots[sl]
        base = tile_base + c * _CHUNK
        pltpu.async_copy(ov, pout_hbm.at[pl.ds(base, _CHUNK)], so)

    def wait_pout(sl):
        _, _, _, ov, _, so = slots[sl]
        pltpu.make_async_copy(ov, pout_hbm.at[pl.ds(0, _CHUNK)], so).wait()

    def start_scat(sl):
        _, pfv, _, ov, _, _ = slots[sl]
        pltpu.async_copy(ov, atoms_sh.at[pfv], sem_sc, add=True)

    def wait_scat(sl):
        _, pfv, _, ov, _, _ = slots[sl]
        pltpu.make_async_copy(ov, atoms_sh.at[pfv], sem_sc).wait()

    def vec_compute(sl, off):
        rv, pfv, psv, ov, _, _ = slots[sl]
        pf16 = pfv[pl.ds(off, 16)]
        ps16 = psv[pl.ds(off, 16)]
        r16 = rv[pl.ds(off, 16)]
        wi = plsc.load_gather(spk_v, [pf16 >> 2])
        wj = plsc.load_gather(spk_v, [ps16 >> 2])
        si = (wi >> ((pf16 & 3) << 3)) & 0xFF
        sj = (wj >> ((ps16 & 3) << 3)) & 0xFF
        pidx = si * 90 + sj
        ia = plsc.load_gather(tia_v, [pidx])
        cc = plsc.load_gather(tc_v, [pidx])
        a3 = plsc.load_gather(ta3_v, [pidx])
        b4 = plsc.load_gather(tb4_v, [pidx])
        zz = jnp.float32(_PREFIX) * si.astype(jnp.float32) * sj.astype(jnp.float32)
        x = r16 * ia
        phi = (c0 * jnp.exp(-e0 * x) + c1 * jnp.exp(-e1 * x)
               + c2 * jnp.exp(-e2 * x) + c3 * jnp.exp(-e3 * x))
        t = r16 - jnp.float32(_R_INNER)
        o2 = zz * (phi / r16 + cc + t * t * t * (a3 + b4 * t))
        o1 = zz * cc
        res = jnp.where(r16 > jnp.float32(_R_OUTER), jnp.float32(0.0),
                        jnp.where(r16 < jnp.float32(_R_INNER), o1, o2))
        ov[pl.ds(off, 16)] = res

    def compute(sl):
        def vec_body(i, carry):
            base = i * 128
            for u in range(8):
                vec_compute(sl, base + u * 16)
            return carry
        lax.fori_loop(0, _CHUNK // 128, vec_body, 0)

    def pair_body(g, carry):
        a = 2 * g

        @pl.when(g > 0)
        def _():
            wait_scat(1)

        start_in(a + 1, 1)
        wait_in(0)

        @pl.when(g > 0)
        def _():
            wait_pout(0)

        compute(0)
        start_pout(a, 0)
        start_scat(0)
        wait_in(1)

        @pl.when(g > 0)
        def _():
            wait_pout(1)

        compute(1)
        wait_scat(0)
        start_pout(a + 1, 1)
        start_scat(1)

        @pl.when(g < _N_CHUNKS // 2 - 1)
        def _():
            start_in(a + 2, 0)
        return carry

    start_in(0, 0)
    lax.fori_loop(0, _N_CHUNKS // 2, pair_body, 0)
    wait_scat(1)
    wait_pout(0)
    wait_pout(1)
    plsc.subcore_barrier()
    pltpu.sync_copy(atoms_sh.at[pl.ds(sid * _NODE_SLICE, _NODE_SLICE)],
                    partial_hbm.at[cid, pl.ds(sid * _NODE_SLICE, _NODE_SLICE)])


def _combine_body(p_ref, o_ref):
    o_ref[...] = p_ref[0] + p_ref[1]


@jax.jit
def kernel(r, pair_first, pair_second, species):
    npad = _NE_PAD - _N_EDGES
    r2 = jnp.concatenate([r, jnp.full((npad,), 3.0, jnp.float32)])
    pf2 = jnp.concatenate([pair_first, jnp.zeros((npad,), jnp.int32)])
    ps2 = jnp.concatenate([pair_second, jnp.zeros((npad,), jnp.int32)])
    spv = species.reshape(_SPK_W, 4)
    spk = (spv[:, 0] | (spv[:, 1] << 8) | (spv[:, 2] << 16) | (spv[:, 3] << 24)).astype(jnp.int32)
    zer = jnp.zeros((_NODE_SLICE,), jnp.float32)

    mesh = plsc.VectorSubcoreMesh(core_axis_name="c", subcore_axis_name="s",
                                  num_cores=_NC, num_subcores=_NS)
    zbl = pl.kernel(
        _zbl_body,
        out_type=[
            jax.ShapeDtypeStruct((_NE_PAD,), jnp.float32),
            jax.ShapeDtypeStruct((_NC, _NODES_PAD), jnp.float32),
        ],
        mesh=mesh,
        compiler_params=pltpu.CompilerParams(needs_layout_passes=False),
        scratch_types=[
            pltpu.VMEM((_SPK_W,), jnp.int32),
            pltpu.VMEM((_TBL,), jnp.float32),
            pltpu.VMEM((_TBL,), jnp.float32),
            pltpu.VMEM((_TBL,), jnp.float32),
            pltpu.VMEM((_TBL,), jnp.float32),
            pltpu.VMEM((_CHUNK,), jnp.float32),
            pltpu.VMEM((_CHUNK,), jnp.int32),
            pltpu.VMEM((_CHUNK,), jnp.int32),
            pltpu.VMEM((_CHUNK,), jnp.float32),
            pltpu.VMEM((_CHUNK,), jnp.float32),
            pltpu.VMEM((_CHUNK,), jnp.int32),
            pltpu.VMEM((_CHUNK,), jnp.int32),
            pltpu.VMEM((_CHUNK,), jnp.float32),
            pltpu.VMEM_SHARED((_NODES_PAD,), jnp.float32),
            pltpu.SemaphoreType.DMA,
            pltpu.SemaphoreType.DMA,
            pltpu.SemaphoreType.DMA,
            pltpu.SemaphoreType.DMA,
            pltpu.SemaphoreType.DMA,
        ],
    )
    pair2, partial = zbl(r2, pf2, ps2, spk,
                         jnp.asarray(_T_IA), jnp.asarray(_T_C),
                         jnp.asarray(_T_A3), jnp.asarray(_T_B4), zer)

    atom_pad = pl.pallas_call(
        _combine_body,
        out_shape=jax.ShapeDtypeStruct((784, 128), jnp.float32),
    )(partial.reshape(_NC, 784, 128))

    pair_output = pair2[:_N_EDGES]
    atom_output = atom_pad.reshape(-1)[:_N_NODES]
    return (pair_output, atom_output)


# speedup vs baseline: 355.3248x; 1.7178x over previous
import functools

import numpy as np
import jax
import jax.numpy as jnp
from jax import lax
from jax.experimental import pallas as pl
from jax.experimental.pallas import tpu as pltpu
from jax.experimental.pallas import tpu_sc as plsc

_COEFF = (0.02817, 0.28022, 0.50986, 0.18175)
_EXPO = (0.20162, 0.4029, 0.94229, 3.1998)
_EXPO_A = 0.23
_A0 = 0.4685
_R_INNER = 0.5
_R_OUTER = 2.0
_PREFIX = 14.399645478425668

_N_NODES = 100000
_N_EDGES = 6400000

_NS = 16
_NC = 2
_NW = _NC * _NS
_EDGES_PER_TILE = 200704
_NE_PAD = _NW * _EDGES_PER_TILE
_CHUNK = 2048
_N_CHUNKS = _EDGES_PER_TILE // _CHUNK
_NODES_PAD = 100352
_NODE_SLICE = _NODES_PAD // _NS
_TBL = 8192
_SPK_W = _N_NODES // 4


def _build_tables():
    coeff = np.array(_COEFF, dtype=np.float64)
    expo = np.array(_EXPO, dtype=np.float64)
    s = np.arange(90, dtype=np.float64)
    zp = s ** _EXPO_A
    u = zp[:, None] + zp[None, :]
    u[0, 0] = 1.0
    inv_a = u / _A0
    x = _R_OUTER * inv_a
    e = np.exp(-expo[:, None, None] * x[None])
    phi = (coeff[:, None, None] * e).sum(0)
    dphi = (coeff[:, None, None] * (-expo[:, None, None] * inv_a[None]) * e).sum(0)
    d2phi = (coeff[:, None, None] * (expo[:, None, None] * inv_a[None]) ** 2 * e).sum(0)
    ro = _R_OUTER
    zble = phi / ro
    dz = (1.0 / ro) * (-phi / ro + dphi)
    d2z = (1.0 / ro) * (d2phi - 2.0 * dphi / ro + 2.0 * phi / ro ** 2)
    tc = ro - _R_INNER
    c = -zble + tc / 2.0 * dz - (1.0 / 12.0) * tc ** 2 * d2z
    b4 = (2.0 * dz - tc * d2z) / tc ** 3 / 4.0
    a3 = (-3.0 * dz + tc * d2z) / tc ** 2 / 3.0

    def pad(t):
        out = np.zeros((_TBL,), dtype=np.float32)
        out[: 90 * 90] = t.astype(np.float32).reshape(-1)
        return out

    return pad(inv_a), pad(c), pad(a3), pad(b4)


_T_IA, _T_C, _T_A3, _T_B4 = _build_tables()


def _zbl_body(r_hbm, pf_hbm, ps_hbm, spk_hbm, tia_hbm, tc_hbm, ta3_hbm, tb4_hbm,
              zer_hbm, pout_hbm, partial_hbm,
              spk_v, tia_v, tc_v, ta3_v, tb4_v,
              r_v0, pf_v0, ps_v0, out_v0, r_v1, pf_v1, ps_v1, out_v1, atoms_sh,
              sem_in0, sem_in1, sem_po0, sem_po1, sem_sc):
    cid = lax.axis_index("c")
    sid = lax.axis_index("s")
    wid = cid * _NS + sid
    tile_base = wid * _EDGES_PER_TILE

    pltpu.sync_copy(spk_hbm, spk_v)
    pltpu.sync_copy(tia_hbm, tia_v)
    pltpu.sync_copy(tc_hbm, tc_v)
    pltpu.sync_copy(ta3_hbm, ta3_v)
    pltpu.sync_copy(tb4_hbm, tb4_v)
    pltpu.sync_copy(zer_hbm, atoms_sh.at[pl.ds(sid * _NODE_SLICE, _NODE_SLICE)])
    plsc.subcore_barrier()

    c0, c1, c2, c3 = (jnp.float32(v) for v in _COEFF)
    e0, e1, e2, e3 = (jnp.float32(v) for v in _EXPO)

    slots = ((r_v0, pf_v0, ps_v0, out_v0, sem_in0, sem_po0),
             (r_v1, pf_v1, ps_v1, out_v1, sem_in1, sem_po1))

    def start_in(c, sl):
        rv, pfv, psv, _, si, _ = slots[sl]
        base = tile_base + c * _CHUNK
        pltpu.async_copy(r_hbm.at[pl.ds(base, _CHUNK)], rv, si)
        pltpu.async_copy(pf_hbm.at[pl.ds(base, _CHUNK)], pfv, si)
        pltpu.async_copy(ps_hbm.at[pl.ds(base, _CHUNK)], psv, si)

    def wait_in(sl):
        rv, pfv, psv, _, si, _ = slots[sl]
        pltpu.make_async_copy(r_hbm.at[pl.ds(0, _CHUNK)], rv, si).wait()
        pltpu.make_async_copy(pf_hbm.at[pl.ds(0, _CHUNK)], pfv, si).wait()
        pltpu.make_async_copy(ps_hbm.at[pl.ds(0, _CHUNK)], psv, si).wait()

    def start_pout(c, sl):
        _, _, _, ov, _, so = slots[sl]
        base = tile_base + c * _CHUNK
        pltpu.async_copy(ov, pout_hbm.at[pl.ds(base, _CHUNK)], so)

    def wait_pout(sl):
        _, _, _, ov, _, so = slots[sl]
        pltpu.make_async_copy(ov, pout_hbm.at[pl.ds(0, _CHUNK)], so).wait()

    def start_scat(sl):
        _, pfv, _, ov, _, _ = slots[sl]
        pltpu.async_copy(ov, atoms_sh.at[pfv], sem_sc, add=True)

    def wait_scat(sl):
        _, pfv, _, ov, _, _ = slots[sl]
        pltpu.make_async_copy(ov, atoms_sh.at[pfv], sem_sc).wait()

    def vec_compute(sl, off):
        rv, pfv, psv, ov, _, _ = slots[sl]
        pf16 = pfv[pl.ds(off, 16)]
        ps16 = psv[pl.ds(off, 16)]
        r16 = rv[pl.ds(off, 16)]
        wi = plsc.load_gather(spk_v, [pf16 >> 2])
        wj = plsc.load_gather(spk_v, [ps16 >> 2])
        si = (wi >> ((pf16 & 3) << 3)) & 0xFF
        sj = (wj >> ((ps16 & 3) << 3)) & 0xFF
        pidx = si * 90 + sj
        ia = plsc.load_gather(tia_v, [pidx])
        cc = plsc.load_gather(tc_v, [pidx])
        a3 = plsc.load_gather(ta3_v, [pidx])
        b4 = plsc.load_gather(tb4_v, [pidx])
        zz = jnp.float32(_PREFIX) * si.astype(jnp.float32) * sj.astype(jnp.float32)
        x = r16 * ia
        phi = (c0 * jnp.exp(-e0 * x) + c1 * jnp.exp(-e1 * x)
               + c2 * jnp.exp(-e2 * x) + c3 * jnp.exp(-e3 * x))
        t = r16 - jnp.float32(_R_INNER)
        o2 = zz * (phi / r16 + cc + t * t * t * (a3 + b4 * t))
        o1 = zz * cc
        res = jnp.where(r16 > jnp.float32(_R_OUTER), jnp.float32(0.0),
                        jnp.where(r16 < jnp.float32(_R_INNER), o1, o2))
        ov[pl.ds(off, 16)] = res

    def compute(sl):
        @plsc.parallel_loop(0, _CHUNK // 16, step=1, unroll=8)
        def _(i):
            vec_compute(sl, i * 16)

    def pair_body(g, carry):
        a = 2 * g

        @pl.when(g > 0)
        def _():
            wait_scat(1)

        start_in(a + 1, 1)
        wait_in(0)

        @pl.when(g > 0)
        def _():
            wait_pout(0)

        compute(0)
        start_pout(a, 0)
        start_scat(0)
        wait_in(1)

        @pl.when(g > 0)
        def _():
            wait_pout(1)

        compute(1)
        wait_scat(0)
        start_pout(a + 1, 1)
        start_scat(1)

        @pl.when(g < _N_CHUNKS // 2 - 1)
        def _():
            start_in(a + 2, 0)
        return carry

    start_in(0, 0)
    lax.fori_loop(0, _N_CHUNKS // 2, pair_body, 0)
    wait_scat(1)
    wait_pout(0)
    wait_pout(1)
    plsc.subcore_barrier()
    pltpu.sync_copy(atoms_sh.at[pl.ds(sid * _NODE_SLICE, _NODE_SLICE)],
                    partial_hbm.at[cid, pl.ds(sid * _NODE_SLICE, _NODE_SLICE)])


def _combine_body(p_ref, o_ref):
    o_ref[...] = p_ref[0] + p_ref[1]


@jax.jit
def kernel(r, pair_first, pair_second, species):
    npad = _NE_PAD - _N_EDGES
    r2 = jnp.concatenate([r, jnp.full((npad,), 3.0, jnp.float32)])
    pf2 = jnp.concatenate([pair_first, jnp.zeros((npad,), jnp.int32)])
    ps2 = jnp.concatenate([pair_second, jnp.zeros((npad,), jnp.int32)])
    spv = species.reshape(_SPK_W, 4)
    spk = (spv[:, 0] | (spv[:, 1] << 8) | (spv[:, 2] << 16) | (spv[:, 3] << 24)).astype(jnp.int32)
    zer = jnp.zeros((_NODE_SLICE,), jnp.float32)

    mesh = plsc.VectorSubcoreMesh(core_axis_name="c", subcore_axis_name="s",
                                  num_cores=_NC, num_subcores=_NS)
    zbl = pl.kernel(
        _zbl_body,
        out_type=[
            jax.ShapeDtypeStruct((_NE_PAD,), jnp.float32),
            jax.ShapeDtypeStruct((_NC, _NODES_PAD), jnp.float32),
        ],
        mesh=mesh,
        compiler_params=pltpu.CompilerParams(needs_layout_passes=False),
        scratch_types=[
            pltpu.VMEM((_SPK_W,), jnp.int32),
            pltpu.VMEM((_TBL,), jnp.float32),
            pltpu.VMEM((_TBL,), jnp.float32),
            pltpu.VMEM((_TBL,), jnp.float32),
            pltpu.VMEM((_TBL,), jnp.float32),
            pltpu.VMEM((_CHUNK,), jnp.float32),
            pltpu.VMEM((_CHUNK,), jnp.int32),
            pltpu.VMEM((_CHUNK,), jnp.int32),
            pltpu.VMEM((_CHUNK,), jnp.float32),
            pltpu.VMEM((_CHUNK,), jnp.float32),
            pltpu.VMEM((_CHUNK,), jnp.int32),
            pltpu.VMEM((_CHUNK,), jnp.int32),
            pltpu.VMEM((_CHUNK,), jnp.float32),
            pltpu.VMEM_SHARED((_NODES_PAD,), jnp.float32),
            pltpu.SemaphoreType.DMA,
            pltpu.SemaphoreType.DMA,
            pltpu.SemaphoreType.DMA,
            pltpu.SemaphoreType.DMA,
            pltpu.SemaphoreType.DMA,
        ],
    )
    pair2, partial = zbl(r2, pf2, ps2, spk,
                         jnp.asarray(_T_IA), jnp.asarray(_T_C),
                         jnp.asarray(_T_A3), jnp.asarray(_T_B4), zer)

    atom_pad = pl.pallas_call(
        _combine_body,
        out_shape=jax.ShapeDtypeStruct((784, 128), jnp.float32),
    )(partial.reshape(_NC, 784, 128))

    pair_output = pair2[:_N_EDGES]
    atom_output = atom_pad.reshape(-1)[:_N_NODES]
    return (pair_output, atom_output)


# speedup vs baseline: 373.9498x; 1.0524x over previous
import functools

import numpy as np
import jax
import jax.numpy as jnp
from jax import lax
from jax.experimental import pallas as pl
from jax.experimental.pallas import tpu as pltpu
from jax.experimental.pallas import tpu_sc as plsc

_COEFF = (0.02817, 0.28022, 0.50986, 0.18175)
_EXPO = (0.20162, 0.4029, 0.94229, 3.1998)
_EXPO_A = 0.23
_A0 = 0.4685
_R_INNER = 0.5
_R_OUTER = 2.0
_PREFIX = 14.399645478425668

_N_NODES = 100000
_N_EDGES = 6400000

_NS = 16
_NC = 2
_NW = _NC * _NS
_EDGES_PER_TILE = 200704
_NE_PAD = _NW * _EDGES_PER_TILE
_CHUNK = 2048
_N_CHUNKS = _EDGES_PER_TILE // _CHUNK
_NODES_PAD = 100352
_NODE_SLICE = _NODES_PAD // _NS
_TBL = 8192
_SPK_W = _N_NODES // 4


def _build_tables():
    coeff = np.array(_COEFF, dtype=np.float64)
    expo = np.array(_EXPO, dtype=np.float64)
    s = np.arange(90, dtype=np.float64)
    zp = s ** _EXPO_A
    u = zp[:, None] + zp[None, :]
    u[0, 0] = 1.0
    inv_a = u / _A0
    x = _R_OUTER * inv_a
    e = np.exp(-expo[:, None, None] * x[None])
    phi = (coeff[:, None, None] * e).sum(0)
    dphi = (coeff[:, None, None] * (-expo[:, None, None] * inv_a[None]) * e).sum(0)
    d2phi = (coeff[:, None, None] * (expo[:, None, None] * inv_a[None]) ** 2 * e).sum(0)
    ro = _R_OUTER
    zble = phi / ro
    dz = (1.0 / ro) * (-phi / ro + dphi)
    d2z = (1.0 / ro) * (d2phi - 2.0 * dphi / ro + 2.0 * phi / ro ** 2)
    tc = ro - _R_INNER
    c = -zble + tc / 2.0 * dz - (1.0 / 12.0) * tc ** 2 * d2z
    b4 = (2.0 * dz - tc * d2z) / tc ** 3 / 4.0
    a3 = (-3.0 * dz + tc * d2z) / tc ** 2 / 3.0

    def pad(t):
        out = np.zeros((_TBL,), dtype=np.float32)
        out[: 90 * 90] = t.astype(np.float32).reshape(-1)
        return out

    return pad(inv_a), pad(c), pad(a3), pad(b4)


_T_IA, _T_C, _T_A3, _T_B4 = _build_tables()


def _zbl_body(r_hbm, pf_hbm, ps_hbm, spk_hbm, tia_hbm, tc_hbm, ta3_hbm, tb4_hbm,
              zer_hbm, pout_hbm, partial_hbm,
              spk_v, tia_v, tc_v, ta3_v, tb4_v,
              r_v0, pf_v0, ps_v0, out_v0, r_v1, pf_v1, ps_v1, out_v1, atoms_sh,
              sem_in0, sem_in1, sem_po0, sem_po1, sem_sc):
    cid = lax.axis_index("c")
    sid = lax.axis_index("s")
    wid = cid * _NS + sid
    tile_base = wid * _EDGES_PER_TILE

    pltpu.sync_copy(spk_hbm, spk_v)
    pltpu.sync_copy(tia_hbm, tia_v)
    pltpu.sync_copy(tc_hbm, tc_v)
    pltpu.sync_copy(ta3_hbm, ta3_v)
    pltpu.sync_copy(tb4_hbm, tb4_v)
    pltpu.sync_copy(zer_hbm, atoms_sh.at[pl.ds(sid * _NODE_SLICE, _NODE_SLICE)])
    plsc.subcore_barrier()

    c0, c1, c2, c3 = (jnp.float32(v) for v in _COEFF)
    e0, e1, e2, e3 = (jnp.float32(v) for v in _EXPO)

    slots = ((r_v0, pf_v0, ps_v0, out_v0, sem_in0, sem_po0),
             (r_v1, pf_v1, ps_v1, out_v1, sem_in1, sem_po1))

    def start_in(c, sl):
        rv, pfv, psv, _, si, _ = slots[sl]
        base = tile_base + c * _CHUNK
        pltpu.async_copy(r_hbm.at[pl.ds(base, _CHUNK)], rv, si)
        pltpu.async_copy(pf_hbm.at[pl.ds(base, _CHUNK)], pfv, si)
        pltpu.async_copy(ps_hbm.at[pl.ds(base, _CHUNK)], psv, si)

    def wait_in(sl):
        rv, pfv, psv, _, si, _ = slots[sl]
        pltpu.make_async_copy(r_hbm.at[pl.ds(0, _CHUNK)], rv, si).wait()
        pltpu.make_async_copy(pf_hbm.at[pl.ds(0, _CHUNK)], pfv, si).wait()
        pltpu.make_async_copy(ps_hbm.at[pl.ds(0, _CHUNK)], psv, si).wait()

    def start_pout(c, sl):
        _, _, _, ov, _, so = slots[sl]
        base = tile_base + c * _CHUNK
        pltpu.async_copy(ov, pout_hbm.at[pl.ds(base, _CHUNK)], so)

    def wait_pout(sl):
        _, _, _, ov, _, so = slots[sl]
        pltpu.make_async_copy(ov, pout_hbm.at[pl.ds(0, _CHUNK)], so).wait()

    def start_scat(sl):
        _, pfv, _, ov, _, _ = slots[sl]
        pltpu.async_copy(ov, atoms_sh.at[pfv], sem_sc, add=True)

    def wait_scat(sl):
        _, pfv, _, ov, _, _ = slots[sl]
        pltpu.make_async_copy(ov, atoms_sh.at[pfv], sem_sc).wait()

    def vec_compute(sl, off):
        rv, pfv, psv, ov, _, _ = slots[sl]
        pf16 = pfv[pl.ds(off, 16)]
        ps16 = psv[pl.ds(off, 16)]
        r16 = rv[pl.ds(off, 16)]
        wi = plsc.load_gather(spk_v, [pf16 >> 2])
        wj = plsc.load_gather(spk_v, [ps16 >> 2])
        si = (wi >> ((pf16 & 3) << 3)) & 0xFF
        sj = (wj >> ((ps16 & 3) << 3)) & 0xFF
        pidx = si * 90 + sj
        ia = plsc.load_gather(tia_v, [pidx])
        cc = plsc.load_gather(tc_v, [pidx])
        a3 = plsc.load_gather(ta3_v, [pidx])
        b4 = plsc.load_gather(tb4_v, [pidx])
        zz = jnp.float32(_PREFIX) * si.astype(jnp.float32) * sj.astype(jnp.float32)
        x = r16 * ia
        phi = (c0 * jnp.exp(-e0 * x) + c1 * jnp.exp(-e1 * x)
               + c2 * jnp.exp(-e2 * x) + c3 * jnp.exp(-e3 * x))
        t = r16 - jnp.float32(_R_INNER)
        o2 = zz * (phi / r16 + cc + t * t * t * (a3 + b4 * t))
        o1 = zz * cc
        res = jnp.where(r16 > jnp.float32(_R_OUTER), jnp.float32(0.0),
                        jnp.where(r16 < jnp.float32(_R_INNER), o1, o2))
        ov[pl.ds(off, 16)] = res

    def compute(sl):
        @plsc.parallel_loop(0, _CHUNK // 16, step=1, unroll=16)
        def _(i):
            vec_compute(sl, i * 16)

    def pair_body(g, carry):
        a = 2 * g

        @pl.when(g > 0)
        def _():
            wait_scat(1)

        start_in(a + 1, 1)
        wait_in(0)

        @pl.when(g > 0)
        def _():
            wait_pout(0)

        compute(0)
        start_pout(a, 0)
        start_scat(0)
        wait_in(1)

        @pl.when(g > 0)
        def _():
            wait_pout(1)

        compute(1)
        wait_scat(0)
        start_pout(a + 1, 1)
        start_scat(1)

        @pl.when(g < _N_CHUNKS // 2 - 1)
        def _():
            start_in(a + 2, 0)
        return carry

    start_in(0, 0)
    lax.fori_loop(0, _N_CHUNKS // 2, pair_body, 0)
    wait_scat(1)
    wait_pout(0)
    wait_pout(1)
    plsc.subcore_barrier()
    pltpu.sync_copy(atoms_sh.at[pl.ds(sid * _NODE_SLICE, _NODE_SLICE)],
                    partial_hbm.at[cid, pl.ds(sid * _NODE_SLICE, _NODE_SLICE)])


def _combine_body(p_ref, o_ref):
    o_ref[...] = p_ref[0] + p_ref[1]


@jax.jit
def kernel(r, pair_first, pair_second, species):
    npad = _NE_PAD - _N_EDGES
    r2 = jnp.concatenate([r, jnp.full((npad,), 3.0, jnp.float32)])
    pf2 = jnp.concatenate([pair_first, jnp.zeros((npad,), jnp.int32)])
    ps2 = jnp.concatenate([pair_second, jnp.zeros((npad,), jnp.int32)])
    spv = species.reshape(_SPK_W, 4)
    spk = (spv[:, 0] | (spv[:, 1] << 8) | (spv[:, 2] << 16) | (spv[:, 3] << 24)).astype(jnp.int32)
    zer = jnp.zeros((_NODE_SLICE,), jnp.float32)

    mesh = plsc.VectorSubcoreMesh(core_axis_name="c", subcore_axis_name="s",
                                  num_cores=_NC, num_subcores=_NS)
    zbl = pl.kernel(
        _zbl_body,
        out_type=[
            jax.ShapeDtypeStruct((_NE_PAD,), jnp.float32),
            jax.ShapeDtypeStruct((_NC, _NODES_PAD), jnp.float32),
        ],
        mesh=mesh,
        compiler_params=pltpu.CompilerParams(needs_layout_passes=False),
        scratch_types=[
            pltpu.VMEM((_SPK_W,), jnp.int32),
            pltpu.VMEM((_TBL,), jnp.float32),
            pltpu.VMEM((_TBL,), jnp.float32),
            pltpu.VMEM((_TBL,), jnp.float32),
            pltpu.VMEM((_TBL,), jnp.float32),
            pltpu.VMEM((_CHUNK,), jnp.float32),
            pltpu.VMEM((_CHUNK,), jnp.int32),
            pltpu.VMEM((_CHUNK,), jnp.int32),
            pltpu.VMEM((_CHUNK,), jnp.float32),
            pltpu.VMEM((_CHUNK,), jnp.float32),
            pltpu.VMEM((_CHUNK,), jnp.int32),
            pltpu.VMEM((_CHUNK,), jnp.int32),
            pltpu.VMEM((_CHUNK,), jnp.float32),
            pltpu.VMEM_SHARED((_NODES_PAD,), jnp.float32),
            pltpu.SemaphoreType.DMA,
            pltpu.SemaphoreType.DMA,
            pltpu.SemaphoreType.DMA,
            pltpu.SemaphoreType.DMA,
            pltpu.SemaphoreType.DMA,
        ],
    )
    pair2, partial = zbl(r2, pf2, ps2, spk,
                         jnp.asarray(_T_IA), jnp.asarray(_T_C),
                         jnp.asarray(_T_A3), jnp.asarray(_T_B4), zer)

    atom_pad = pl.pallas_call(
        _combine_body,
        out_shape=jax.ShapeDtypeStruct((784, 128), jnp.float32),
    )(partial.reshape(_NC, 784, 128))

    pair_output = pair2[:_N_EDGES]
    atom_output = atom_pad.reshape(-1)[:_N_NODES]
    return (pair_output, atom_output)


# speedup vs baseline: 375.4500x; 1.0040x over previous
import functools

import numpy as np
import jax
import jax.numpy as jnp
from jax import lax
from jax.experimental import pallas as pl
from jax.experimental.pallas import tpu as pltpu
from jax.experimental.pallas import tpu_sc as plsc

_COEFF = (0.02817, 0.28022, 0.50986, 0.18175)
_EXPO = (0.20162, 0.4029, 0.94229, 3.1998)
_EXPO_A = 0.23
_A0 = 0.4685
_R_INNER = 0.5
_R_OUTER = 2.0
_PREFIX = 14.399645478425668

_N_NODES = 100000
_N_EDGES = 6400000

_NS = 16
_NC = 2
_NW = _NC * _NS
_EDGES_PER_TILE = 200704
_NE_PAD = _NW * _EDGES_PER_TILE
_CHUNK = 6272
_N_CHUNKS = _EDGES_PER_TILE // _CHUNK
_NODES_PAD = 100352
_NODE_SLICE = _NODES_PAD // _NS
_TBL = 8192
_SPK_W = _N_NODES // 4


def _build_tables():
    coeff = np.array(_COEFF, dtype=np.float64)
    expo = np.array(_EXPO, dtype=np.float64)
    s = np.arange(90, dtype=np.float64)
    zp = s ** _EXPO_A
    u = zp[:, None] + zp[None, :]
    u[0, 0] = 1.0
    inv_a = u / _A0
    x = _R_OUTER * inv_a
    e = np.exp(-expo[:, None, None] * x[None])
    phi = (coeff[:, None, None] * e).sum(0)
    dphi = (coeff[:, None, None] * (-expo[:, None, None] * inv_a[None]) * e).sum(0)
    d2phi = (coeff[:, None, None] * (expo[:, None, None] * inv_a[None]) ** 2 * e).sum(0)
    ro = _R_OUTER
    zble = phi / ro
    dz = (1.0 / ro) * (-phi / ro + dphi)
    d2z = (1.0 / ro) * (d2phi - 2.0 * dphi / ro + 2.0 * phi / ro ** 2)
    tc = ro - _R_INNER
    c = -zble + tc / 2.0 * dz - (1.0 / 12.0) * tc ** 2 * d2z
    b4 = (2.0 * dz - tc * d2z) / tc ** 3 / 4.0
    a3 = (-3.0 * dz + tc * d2z) / tc ** 2 / 3.0

    def pad(t):
        out = np.zeros((_TBL,), dtype=np.float32)
        out[: 90 * 90] = t.astype(np.float32).reshape(-1)
        return out

    return pad(inv_a), pad(c), pad(a3), pad(b4)


_T_IA, _T_C, _T_A3, _T_B4 = _build_tables()


def _zbl_body(r_hbm, pf_hbm, ps_hbm, spk_hbm, tia_hbm, tc_hbm, ta3_hbm, tb4_hbm,
              zer_hbm, pout_hbm, partial_hbm,
              spk_v, tia_v, tc_v, ta3_v, tb4_v,
              r_v0, pf_v0, ps_v0, out_v0, r_v1, pf_v1, ps_v1, out_v1, atoms_sh,
              sem_in0, sem_in1, sem_po0, sem_po1, sem_sc):
    cid = lax.axis_index("c")
    sid = lax.axis_index("s")
    wid = cid * _NS + sid
    tile_base = wid * _EDGES_PER_TILE

    pltpu.sync_copy(spk_hbm, spk_v)
    pltpu.sync_copy(tia_hbm, tia_v)
    pltpu.sync_copy(tc_hbm, tc_v)
    pltpu.sync_copy(ta3_hbm, ta3_v)
    pltpu.sync_copy(tb4_hbm, tb4_v)
    pltpu.sync_copy(zer_hbm, atoms_sh.at[pl.ds(sid * _NODE_SLICE, _NODE_SLICE)])
    plsc.subcore_barrier()

    c0, c1, c2, c3 = (jnp.float32(v) for v in _COEFF)
    e0, e1, e2, e3 = (jnp.float32(v) for v in _EXPO)

    slots = ((r_v0, pf_v0, ps_v0, out_v0, sem_in0, sem_po0),
             (r_v1, pf_v1, ps_v1, out_v1, sem_in1, sem_po1))

    def start_in(c, sl):
        rv, pfv, psv, _, si, _ = slots[sl]
        base = tile_base + c * _CHUNK
        pltpu.async_copy(r_hbm.at[pl.ds(base, _CHUNK)], rv, si)
        pltpu.async_copy(pf_hbm.at[pl.ds(base, _CHUNK)], pfv, si)
        pltpu.async_copy(ps_hbm.at[pl.ds(base, _CHUNK)], psv, si)

    def wait_in(sl):
        rv, pfv, psv, _, si, _ = slots[sl]
        pltpu.make_async_copy(r_hbm.at[pl.ds(0, _CHUNK)], rv, si).wait()
        pltpu.make_async_copy(pf_hbm.at[pl.ds(0, _CHUNK)], pfv, si).wait()
        pltpu.make_async_copy(ps_hbm.at[pl.ds(0, _CHUNK)], psv, si).wait()

    def start_pout(c, sl):
        _, _, _, ov, _, so = slots[sl]
        base = tile_base + c * _CHUNK
        pltpu.async_copy(ov, pout_hbm.at[pl.ds(base, _CHUNK)], so)

    def wait_pout(sl):
        _, _, _, ov, _, so = slots[sl]
        pltpu.make_async_copy(ov, pout_hbm.at[pl.ds(0, _CHUNK)], so).wait()

    def start_scat(sl):
        _, pfv, _, ov, _, _ = slots[sl]
        pltpu.async_copy(ov, atoms_sh.at[pfv], sem_sc, add=True)

    def wait_scat(sl):
        _, pfv, _, ov, _, _ = slots[sl]
        pltpu.make_async_copy(ov, atoms_sh.at[pfv], sem_sc).wait()

    def vec_compute(sl, off):
        rv, pfv, psv, ov, _, _ = slots[sl]
        pf16 = pfv[pl.ds(off, 16)]
        ps16 = psv[pl.ds(off, 16)]
        r16 = rv[pl.ds(off, 16)]
        wi = plsc.load_gather(spk_v, [pf16 >> 2])
        wj = plsc.load_gather(spk_v, [ps16 >> 2])
        si = (wi >> ((pf16 & 3) << 3)) & 0xFF
        sj = (wj >> ((ps16 & 3) << 3)) & 0xFF
        pidx = si * 90 + sj
        ia = plsc.load_gather(tia_v, [pidx])
        cc = plsc.load_gather(tc_v, [pidx])
        a3 = plsc.load_gather(ta3_v, [pidx])
        b4 = plsc.load_gather(tb4_v, [pidx])
        zz = jnp.float32(_PREFIX) * si.astype(jnp.float32) * sj.astype(jnp.float32)
        x = r16 * ia
        phi = (c0 * jnp.exp(-e0 * x) + c1 * jnp.exp(-e1 * x)
               + c2 * jnp.exp(-e2 * x) + c3 * jnp.exp(-e3 * x))
        t = r16 - jnp.float32(_R_INNER)
        o2 = zz * (phi / r16 + cc + t * t * t * (a3 + b4 * t))
        o1 = zz * cc
        res = jnp.where(r16 > jnp.float32(_R_OUTER), jnp.float32(0.0),
                        jnp.where(r16 < jnp.float32(_R_INNER), o1, o2))
        ov[pl.ds(off, 16)] = res

    def compute(sl):
        @plsc.parallel_loop(0, _CHUNK // 16, step=1, unroll=16)
        def _(i):
            vec_compute(sl, i * 16)

    def pair_body(g, carry):
        a = 2 * g

        @pl.when(g > 0)
        def _():
            wait_scat(1)

        start_in(a + 1, 1)
        wait_in(0)

        @pl.when(g > 0)
        def _():
            wait_pout(0)

        compute(0)
        start_pout(a, 0)
        start_scat(0)
        wait_in(1)

        @pl.when(g > 0)
        def _():
            wait_pout(1)

        compute(1)
        wait_scat(0)
        start_pout(a + 1, 1)
        start_scat(1)

        @pl.when(g < _N_CHUNKS // 2 - 1)
        def _():
            start_in(a + 2, 0)
        return carry

    start_in(0, 0)
    lax.fori_loop(0, _N_CHUNKS // 2, pair_body, 0)
    wait_scat(1)
    wait_pout(0)
    wait_pout(1)
    plsc.subcore_barrier()
    pltpu.sync_copy(atoms_sh.at[pl.ds(sid * _NODE_SLICE, _NODE_SLICE)],
                    partial_hbm.at[cid, pl.ds(sid * _NODE_SLICE, _NODE_SLICE)])


def _combine_body(p_ref, o_ref):
    o_ref[...] = p_ref[0] + p_ref[1]


@jax.jit
def kernel(r, pair_first, pair_second, species):
    npad = _NE_PAD - _N_EDGES
    r2 = jnp.concatenate([r, jnp.full((npad,), 3.0, jnp.float32)])
    pf2 = jnp.concatenate([pair_first, jnp.zeros((npad,), jnp.int32)])
    ps2 = jnp.concatenate([pair_second, jnp.zeros((npad,), jnp.int32)])
    spv = species.reshape(_SPK_W, 4)
    spk = (spv[:, 0] | (spv[:, 1] << 8) | (spv[:, 2] << 16) | (spv[:, 3] << 24)).astype(jnp.int32)
    zer = jnp.zeros((_NODE_SLICE,), jnp.float32)

    mesh = plsc.VectorSubcoreMesh(core_axis_name="c", subcore_axis_name="s",
                                  num_cores=_NC, num_subcores=_NS)
    zbl = pl.kernel(
        _zbl_body,
        out_type=[
            jax.ShapeDtypeStruct((_NE_PAD,), jnp.float32),
            jax.ShapeDtypeStruct((_NC, _NODES_PAD), jnp.float32),
        ],
        mesh=mesh,
        compiler_params=pltpu.CompilerParams(needs_layout_passes=False),
        scratch_types=[
            pltpu.VMEM((_SPK_W,), jnp.int32),
            pltpu.VMEM((_TBL,), jnp.float32),
            pltpu.VMEM((_TBL,), jnp.float32),
            pltpu.VMEM((_TBL,), jnp.float32),
            pltpu.VMEM((_TBL,), jnp.float32),
            pltpu.VMEM((_CHUNK,), jnp.float32),
            pltpu.VMEM((_CHUNK,), jnp.int32),
            pltpu.VMEM((_CHUNK,), jnp.int32),
            pltpu.VMEM((_CHUNK,), jnp.float32),
            pltpu.VMEM((_CHUNK,), jnp.float32),
            pltpu.VMEM((_CHUNK,), jnp.int32),
            pltpu.VMEM((_CHUNK,), jnp.int32),
            pltpu.VMEM((_CHUNK,), jnp.float32),
            pltpu.VMEM_SHARED((_NODES_PAD,), jnp.float32),
            pltpu.SemaphoreType.DMA,
            pltpu.SemaphoreType.DMA,
            pltpu.SemaphoreType.DMA,
            pltpu.SemaphoreType.DMA,
            pltpu.SemaphoreType.DMA,
        ],
    )
    pair2, partial = zbl(r2, pf2, ps2, spk,
                         jnp.asarray(_T_IA), jnp.asarray(_T_C),
                         jnp.asarray(_T_A3), jnp.asarray(_T_B4), zer)

    atom_pad = pl.pallas_call(
        _combine_body,
        out_shape=jax.ShapeDtypeStruct((784, 128), jnp.float32),
    )(partial.reshape(_NC, 784, 128))

    pair_output = pair2[:_N_EDGES]
    atom_output = atom_pad.reshape(-1)[:_N_NODES]
    return (pair_output, atom_output)


# speedup vs baseline: 442.1645x; 1.1777x over previous
import functools

import numpy as np
import jax
import jax.numpy as jnp
from jax import lax
from jax.experimental import pallas as pl
from jax.experimental.pallas import tpu as pltpu
from jax.experimental.pallas import tpu_sc as plsc

_COEFF = (0.02817, 0.28022, 0.50986, 0.18175)
_EXPO = (0.20162, 0.4029, 0.94229, 3.1998)
_EXPO_A = 0.23
_A0 = 0.4685
_R_INNER = 0.5
_R_OUTER = 2.0
_PREFIX = 14.399645478425668

_N_NODES = 100000
_N_EDGES = 6400000

_NS = 16
_NC = 2
_NW = _NC * _NS
_EDGES_PER_TILE = _N_EDGES // _NW
_CHUNK = 4000
_N_CHUNKS = _EDGES_PER_TILE // _CHUNK
_NODES_PAD = 100352
_NODE_SLICE = _NODES_PAD // _NS
_TBL = 8192
_SPK_W = _N_NODES // 4


def _build_tables():
    coeff = np.array(_COEFF, dtype=np.float64)
    expo = np.array(_EXPO, dtype=np.float64)
    s = np.arange(90, dtype=np.float64)
    zp = s ** _EXPO_A
    u = zp[:, None] + zp[None, :]
    u[0, 0] = 1.0
    inv_a = u / _A0
    x = _R_OUTER * inv_a
    e = np.exp(-expo[:, None, None] * x[None])
    phi = (coeff[:, None, None] * e).sum(0)
    dphi = (coeff[:, None, None] * (-expo[:, None, None] * inv_a[None]) * e).sum(0)
    d2phi = (coeff[:, None, None] * (expo[:, None, None] * inv_a[None]) ** 2 * e).sum(0)
    ro = _R_OUTER
    zble = phi / ro
    dz = (1.0 / ro) * (-phi / ro + dphi)
    d2z = (1.0 / ro) * (d2phi - 2.0 * dphi / ro + 2.0 * phi / ro ** 2)
    tc = ro - _R_INNER
    c = -zble + tc / 2.0 * dz - (1.0 / 12.0) * tc ** 2 * d2z
    b4 = (2.0 * dz - tc * d2z) / tc ** 3 / 4.0
    a3 = (-3.0 * dz + tc * d2z) / tc ** 2 / 3.0

    def pad(t):
        out = np.zeros((_TBL,), dtype=np.float32)
        out[: 90 * 90] = t.astype(np.float32).reshape(-1)
        return out

    return pad(inv_a), pad(c), pad(a3), pad(b4)


_T_IA, _T_C, _T_A3, _T_B4 = _build_tables()


def _zbl_body(r_hbm, pf_hbm, ps_hbm, spk_hbm, tia_hbm, tc_hbm, ta3_hbm, tb4_hbm,
              zer_hbm, pout_hbm, partial_hbm,
              spk_v, tia_v, tc_v, ta3_v, tb4_v,
              r_v0, pf_v0, ps_v0, out_v0, r_v1, pf_v1, ps_v1, out_v1, atoms_sh,
              sem_in0, sem_in1, sem_po0, sem_po1, sem_sc):
    cid = lax.axis_index("c")
    sid = lax.axis_index("s")
    wid = cid * _NS + sid
    tile_base = wid * _EDGES_PER_TILE

    pltpu.sync_copy(spk_hbm, spk_v)
    pltpu.sync_copy(tia_hbm, tia_v)
    pltpu.sync_copy(tc_hbm, tc_v)
    pltpu.sync_copy(ta3_hbm, ta3_v)
    pltpu.sync_copy(tb4_hbm, tb4_v)
    pltpu.sync_copy(zer_hbm, atoms_sh.at[pl.ds(sid * _NODE_SLICE, _NODE_SLICE)])
    plsc.subcore_barrier()

    c0, c1, c2, c3 = (jnp.float32(v) for v in _COEFF)
    e0, e1, e2, e3 = (jnp.float32(v) for v in _EXPO)

    slots = ((r_v0, pf_v0, ps_v0, out_v0, sem_in0, sem_po0),
             (r_v1, pf_v1, ps_v1, out_v1, sem_in1, sem_po1))

    def start_in(c, sl):
        rv, pfv, psv, _, si, _ = slots[sl]
        base = tile_base + c * _CHUNK
        pltpu.async_copy(r_hbm.at[pl.ds(base, _CHUNK)], rv, si)
        pltpu.async_copy(pf_hbm.at[pl.ds(base, _CHUNK)], pfv, si)
        pltpu.async_copy(ps_hbm.at[pl.ds(base, _CHUNK)], psv, si)

    def wait_in(sl):
        rv, pfv, psv, _, si, _ = slots[sl]
        pltpu.make_async_copy(r_hbm.at[pl.ds(0, _CHUNK)], rv, si).wait()
        pltpu.make_async_copy(pf_hbm.at[pl.ds(0, _CHUNK)], pfv, si).wait()
        pltpu.make_async_copy(ps_hbm.at[pl.ds(0, _CHUNK)], psv, si).wait()

    def start_pout(c, sl):
        _, _, _, ov, _, so = slots[sl]
        base = tile_base + c * _CHUNK
        pltpu.async_copy(ov, pout_hbm.at[pl.ds(base, _CHUNK)], so)

    def wait_pout(sl):
        _, _, _, ov, _, so = slots[sl]
        pltpu.make_async_copy(ov, pout_hbm.at[pl.ds(0, _CHUNK)], so).wait()

    def start_scat(sl):
        _, pfv, _, ov, _, _ = slots[sl]
        pltpu.async_copy(ov, atoms_sh.at[pfv], sem_sc, add=True)

    def wait_scat(sl):
        _, pfv, _, ov, _, _ = slots[sl]
        pltpu.make_async_copy(ov, atoms_sh.at[pfv], sem_sc).wait()

    def vec_compute(sl, off):
        rv, pfv, psv, ov, _, _ = slots[sl]
        pf16 = pfv[pl.ds(off, 16)]
        ps16 = psv[pl.ds(off, 16)]
        r16 = rv[pl.ds(off, 16)]
        wi = plsc.load_gather(spk_v, [pf16 >> 2])
        wj = plsc.load_gather(spk_v, [ps16 >> 2])
        si = (wi >> ((pf16 & 3) << 3)) & 0xFF
        sj = (wj >> ((ps16 & 3) << 3)) & 0xFF
        pidx = si * 90 + sj
        ia = plsc.load_gather(tia_v, [pidx])
        cc = plsc.load_gather(tc_v, [pidx])
        a3 = plsc.load_gather(ta3_v, [pidx])
        b4 = plsc.load_gather(tb4_v, [pidx])
        zz = jnp.float32(_PREFIX) * si.astype(jnp.float32) * sj.astype(jnp.float32)
        x = r16 * ia
        phi = (c0 * jnp.exp(-e0 * x) + c1 * jnp.exp(-e1 * x)
               + c2 * jnp.exp(-e2 * x) + c3 * jnp.exp(-e3 * x))
        t = r16 - jnp.float32(_R_INNER)
        o2 = zz * (phi / r16 + cc + t * t * t * (a3 + b4 * t))
        o1 = zz * cc
        res = jnp.where(r16 > jnp.float32(_R_OUTER), jnp.float32(0.0),
                        jnp.where(r16 < jnp.float32(_R_INNER), o1, o2))
        ov[pl.ds(off, 16)] = res

    def compute(sl):
        @plsc.parallel_loop(0, _CHUNK // 16, step=1, unroll=10)
        def _(i):
            vec_compute(sl, i * 16)

    def pair_body(g, carry):
        a = 2 * g

        @pl.when(g > 0)
        def _():
            wait_scat(1)

        start_in(a + 1, 1)
        wait_in(0)

        @pl.when(g > 0)
        def _():
            wait_pout(0)

        compute(0)
        start_pout(a, 0)
        start_scat(0)
        wait_in(1)

        @pl.when(g > 0)
        def _():
            wait_pout(1)

        compute(1)
        wait_scat(0)
        start_pout(a + 1, 1)
        start_scat(1)

        @pl.when(g < _N_CHUNKS // 2 - 1)
        def _():
            start_in(a + 2, 0)
        return carry

    start_in(0, 0)
    lax.fori_loop(0, _N_CHUNKS // 2, pair_body, 0)
    wait_scat(1)
    wait_pout(0)
    wait_pout(1)
    plsc.subcore_barrier()
    pltpu.sync_copy(atoms_sh.at[pl.ds(sid * _NODE_SLICE, _NODE_SLICE)],
                    partial_hbm.at[cid, pl.ds(sid * _NODE_SLICE, _NODE_SLICE)])


def _combine_body(p_ref, o_ref):
    o_ref[...] = p_ref[0] + p_ref[1]


@jax.jit
def kernel(r, pair_first, pair_second, species):
    spv = species.reshape(_SPK_W, 4)
    spk = (spv[:, 0] | (spv[:, 1] << 8) | (spv[:, 2] << 16) | (spv[:, 3] << 24)).astype(jnp.int32)
    zer = jnp.zeros((_NODE_SLICE,), jnp.float32)

    mesh = plsc.VectorSubcoreMesh(core_axis_name="c", subcore_axis_name="s",
                                  num_cores=_NC, num_subcores=_NS)
    zbl = pl.kernel(
        _zbl_body,
        out_type=[
            jax.ShapeDtypeStruct((_N_EDGES,), jnp.float32),
            jax.ShapeDtypeStruct((_NC, _NODES_PAD), jnp.float32),
        ],
        mesh=mesh,
        compiler_params=pltpu.CompilerParams(needs_layout_passes=False),
        scratch_types=[
            pltpu.VMEM((_SPK_W,), jnp.int32),
            pltpu.VMEM((_TBL,), jnp.float32),
            pltpu.VMEM((_TBL,), jnp.float32),
            pltpu.VMEM((_TBL,), jnp.float32),
            pltpu.VMEM((_TBL,), jnp.float32),
            pltpu.VMEM((_CHUNK,), jnp.float32),
            pltpu.VMEM((_CHUNK,), jnp.int32),
            pltpu.VMEM((_CHUNK,), jnp.int32),
            pltpu.VMEM((_CHUNK,), jnp.float32),
            pltpu.VMEM((_CHUNK,), jnp.float32),
            pltpu.VMEM((_CHUNK,), jnp.int32),
            pltpu.VMEM((_CHUNK,), jnp.int32),
            pltpu.VMEM((_CHUNK,), jnp.float32),
            pltpu.VMEM_SHARED((_NODES_PAD,), jnp.float32),
            pltpu.SemaphoreType.DMA,
            pltpu.SemaphoreType.DMA,
            pltpu.SemaphoreType.DMA,
            pltpu.SemaphoreType.DMA,
            pltpu.SemaphoreType.DMA,
        ],
    )
    pair_output, partial = zbl(r, pair_first, pair_second, spk,
                         jnp.asarray(_T_IA), jnp.asarray(_T_C),
                         jnp.asarray(_T_A3), jnp.asarray(_T_B4), zer)

    atom_pad = pl.pallas_call(
        _combine_body,
        out_shape=jax.ShapeDtypeStruct((784, 128), jnp.float32),
    )(partial.reshape(_NC, 784, 128))

    atom_output = atom_pad.reshape(-1)[:_N_NODES]
    return (pair_output, atom_output)
